# baseline (speedup 1.0000x reference)
import math

import jax
import jax.numpy as jnp
from jax.experimental import pallas as pl
from jax.experimental.pallas import tpu as pltpu

T, B = 64, 8
DE = 100
HID = 200
NHID = 100
NLAYERS = 4
N_CLASSES = 6
LAMDA, ALPHA = 0.5, 0.1
N = T * B
GE = 128
DPP = 8
PI = math.pi

_F32 = jnp.float32


def _dot(a, b):
    return jax.lax.dot(a, b, preferred_element_type=_F32)


def _acos(x):
    a = jnp.abs(x)
    p = jnp.float32(-0.0012624911)
    p = p * a + jnp.float32(0.0066700901)
    p = p * a + jnp.float32(-0.0170881256)
    p = p * a + jnp.float32(0.0308918810)
    p = p * a + jnp.float32(-0.0501743046)
    p = p * a + jnp.float32(0.0889789874)
    p = p * a + jnp.float32(-0.2145988016)
    p = p * a + jnp.float32(1.5707963050)
    r = jnp.sqrt(jnp.maximum(1.0 - a, 0.0)) * p
    return jnp.where(x < 0, PI - r, r)


def _sim(c):
    return 1.0 - _acos(jnp.clip(c * 0.99999, -1.0, 1.0)) * (1.0 / PI)


def _encoder_body(u_ref, ua_ref, uv_ref, qm_ref,
                  lawT_ref, lab_ref, lvwT_ref, lvb_ref, llwT_ref, llb_ref,
                  wihT_ref, whhT_ref, bsum_ref, spk_ref,
                  fa_ref, fv_ref, fl_ref,
                  ul_ref, xf_ref, xb_ref, hsf_ref, hsb_ref, out0_ref):
    fa_ref[...] = _dot(ua_ref[...], lawT_ref[...]) + lab_ref[...]
    fv_ref[...] = _dot(uv_ref[...], lvwT_ref[...]) + lvb_ref[...]
    ul_ref[...] = _dot(u_ref[...], llwT_ref[...]) + llb_ref[...]

    for l in range(2):
        xin = ul_ref[...] if l == 0 else out0_ref[...]
        xf_ref[...] = _dot(xin, wihT_ref[l, 0]) + bsum_ref[l, 0]
        xb_ref[...] = _dot(xin, wihT_ref[l, 1]) + bsum_ref[l, 1]
        whf = whhT_ref[l, 0]
        whb = whhT_ref[l, 1]
        bf16 = jnp.bfloat16

        def step(t, carry):
            hf, cf, hb, cb = carry
            gf = xf_ref[pl.ds(t * B, B), :] + _dot(hf.astype(bf16), whf)
            i = jax.nn.sigmoid(gf[:, 0:GE])
            f = jax.nn.sigmoid(gf[:, GE:2 * GE])
            g = jnp.tanh(gf[:, 2 * GE:3 * GE])
            o = jax.nn.sigmoid(gf[:, 3 * GE:4 * GE])
            cf = f * cf + i * g
            hf = o * jnp.tanh(cf)
            hsf_ref[pl.ds(t * B, B), :] = hf

            tb = (T - 1) - t
            gb = xb_ref[pl.ds(tb * B, B), :] + _dot(hb.astype(bf16), whb)
            i = jax.nn.sigmoid(gb[:, 0:GE])
            f = jax.nn.sigmoid(gb[:, GE:2 * GE])
            g = jnp.tanh(gb[:, 2 * GE:3 * GE])
            o = jax.nn.sigmoid(gb[:, 3 * GE:4 * GE])
            cb = f * cb + i * g
            hb = o * jnp.tanh(cb)
            hsb_ref[pl.ds(tb * B, B), :] = hb
            return hf, cf, hb, cb

        z = jnp.zeros((B, GE), _F32)
        jax.lax.fori_loop(0, T, step, (z, z, z, z), unroll=16)
        out0_ref[:, 0:DE] = hsf_ref[:, 0:DE]
        out0_ref[:, DE:HID] = hsb_ref[:, 0:DE]

    q = qm_ref[...]
    sel = q[:, 1:2] > q[:, 0:1]
    emb = jnp.where(sel, spk_ref[1:2, :], spk_ref[0:1, :])
    fl_ref[...] = out0_ref[...] + emb


def _gcn_body(fa_ref, fv_ref, fl_ref, fcwT_ref, fcb_ref, conv_ref,
              wf_ref, wh_ref, smb_ref, out_ref, a_ref, x_ref):
    for d in range(DPP):
        r0 = d * T
        s0 = d * 3 * T
        fs = (fa_ref[pl.ds(r0, T), :], fv_ref[pl.ds(r0, T), :],
              fl_ref[pl.ds(r0, T), :])
        nx = []
        for m in range(3):
            x = fs[m]
            x_ref[pl.ds(s0 + T * m, T), :] = x
            inv = jax.lax.rsqrt(jnp.sum(x * x, axis=1, keepdims=True))
            nx.append(x * inv)

        for m in range(3):
            s = jax.lax.dot_general(nx[m], nx[m], (((1,), (1,)), ((), ())),
                                    preferred_element_type=_F32)
            a_ref[pl.ds(s0 + T * m, T), pl.ds(T * m, T)] = _sim(s)

        row = jax.lax.broadcasted_iota(jnp.int32, (T, T), 0)
        col = jax.lax.broadcasted_iota(jnp.int32, (T, T), 1)
        eye = row == col
        for m in range(3):
            for n in range(m + 1, 3):
                cs = jnp.sum(nx[m] * nx[n], axis=1, keepdims=True)
                tile = jnp.where(eye, _sim(cs), 0.0)
                a_ref[pl.ds(s0 + T * m, T), pl.ds(T * n, T)] = tile
                a_ref[pl.ds(s0 + T * n, T), pl.ds(T * m, T)] = tile

        araw = a_ref[pl.ds(s0, 3 * T), :]
        dcol = jax.lax.rsqrt(jnp.sum(araw, axis=1, keepdims=True))
        drow = jax.lax.rsqrt(jnp.sum(araw, axis=0, keepdims=True))
        a_ref[pl.ds(s0, 3 * T), :] = araw * dcol * drow

    for d in range(DPP):
        r0 = d * T
        s0 = d * 3 * T
        feats = x_ref[pl.ds(s0, 3 * T), :]
        h0 = jax.nn.relu(_dot(feats, fcwT_ref[...]) + fcb_ref[...])
        h = h0
        adj = a_ref[pl.ds(s0, 3 * T), :]
        for i in range(NLAYERS):
            theta = math.log(LAMDA / (i + 1) + 1.0)
            hi = _dot(adj, h)
            mm = (_dot(hi, conv_ref[i, 0:NHID, :])
                  + _dot(h0, conv_ref[i, NHID:2 * NHID, :]))
            r = (1.0 - ALPHA) * hi + ALPHA * h0
            h = jax.nn.relu(theta * mm + (1.0 - theta) * r)

        acc = smb_ref[...]
        for m in range(3):
            fm = jax.nn.relu(feats[T * m:T * (m + 1), :])
            hm = jax.nn.relu(h[T * m:T * (m + 1), :])
            acc = acc + _dot(fm, wf_ref[m]) + _dot(hm, wh_ref[m])
        mx = jnp.max(acc, axis=1, keepdims=True)
        sh = acc - mx
        lse = jnp.log(jnp.sum(jnp.exp(sh), axis=1, keepdims=True))
        out_ref[pl.ds(r0, T), :] = sh - lse


def kernel(U, qmask, U_a, U_v, seq_lengths, lin_a_w, lin_a_b, lin_v_w,
           lin_v_b, lin_l_w, lin_l_b, lstm_wih, lstm_whh, lstm_bih, lstm_bhh,
           spk_emb, gcn_fc_w, gcn_fc_b, conv_w, smax_w, smax_b):
    del seq_lengths

    u_flat = U.reshape(N, -1)
    ua_bt = U_a.transpose(1, 0, 2).reshape(N, -1)
    uv_bt = U_v.transpose(1, 0, 2).reshape(N, -1)
    qm_tb = qmask.reshape(N, 2)
    lawT = lin_a_w.T
    lvwT = lin_v_w.T
    llwT = lin_l_w.T
    lab = lin_a_b.reshape(1, HID)
    lvb = lin_v_b.reshape(1, HID)
    llb = lin_l_b.reshape(1, HID)
    def _pad_gates(w):
        lead = w.shape[:-1]
        w4 = w.reshape(lead + (4, DE))
        pad = [(0, 0)] * len(lead) + [(0, 0), (0, GE - DE)]
        return jnp.pad(w4, pad).reshape(lead + (4 * GE,))

    wihT = _pad_gates(lstm_wih.transpose(0, 1, 3, 2))
    whhT = _pad_gates(lstm_whh.transpose(0, 1, 3, 2))
    whhT = jnp.pad(whhT, ((0, 0), (0, 0), (0, GE - DE), (0, 0)))
    whhT = whhT.astype(jnp.bfloat16)
    bsum = _pad_gates((lstm_bih + lstm_bhh)).reshape(2, 2, 1, 4 * GE)
    fcwT = gcn_fc_w.T
    fcb = gcn_fc_b.reshape(1, NHID)
    smwT = smax_w.T
    wf = jnp.stack([smwT[300 * m:300 * m + HID] for m in range(3)])
    wh = jnp.stack([smwT[300 * m + HID:300 * (m + 1)] for m in range(3)])
    smb = smax_b.reshape(1, N_CLASSES)

    fa, fv, fl_tb = pl.pallas_call(
        _encoder_body,
        out_shape=[
            jax.ShapeDtypeStruct((N, HID), _F32),
            jax.ShapeDtypeStruct((N, HID), _F32),
            jax.ShapeDtypeStruct((N, HID), _F32),
        ],
        scratch_shapes=[
            pltpu.VMEM((N, HID), _F32),
            pltpu.VMEM((N, 4 * GE), _F32),
            pltpu.VMEM((N, 4 * GE), _F32),
            pltpu.VMEM((N, GE), _F32),
            pltpu.VMEM((N, GE), _F32),
            pltpu.VMEM((N, HID), _F32),
        ],
    )(u_flat, ua_bt, uv_bt, qm_tb, lawT, lab, lvwT, lvb, llwT, llb,
      wihT, whhT, bsum, spk_emb)

    fl = fl_tb.reshape(T, B, HID).transpose(1, 0, 2).reshape(N, HID)

    blk = lambda b: (b, 0)
    zero2 = lambda b: (0, 0)
    zero3 = lambda b: (0, 0, 0)
    out = pl.pallas_call(
        _gcn_body,
        grid=(B // DPP,),
        in_specs=[
            pl.BlockSpec((DPP * T, HID), blk),
            pl.BlockSpec((DPP * T, HID), blk),
            pl.BlockSpec((DPP * T, HID), blk),
            pl.BlockSpec((HID, NHID), zero2),
            pl.BlockSpec((1, NHID), zero2),
            pl.BlockSpec((NLAYERS, 2 * NHID, NHID), zero3),
            pl.BlockSpec((3, HID, N_CLASSES), zero3),
            pl.BlockSpec((3, NHID, N_CLASSES), zero3),
            pl.BlockSpec((1, N_CLASSES), zero2),
        ],
        out_specs=pl.BlockSpec((DPP * T, N_CLASSES), blk),
        out_shape=jax.ShapeDtypeStruct((N, N_CLASSES), _F32),
        compiler_params=pltpu.CompilerParams(
            dimension_semantics=("parallel",)),
        scratch_shapes=[
            pltpu.VMEM((DPP * 3 * T, 3 * T), _F32),
            pltpu.VMEM((DPP * 3 * T, HID), _F32),
        ],
    )(fa, fv, fl, fcwT, fcb, conv_w, wf, wh, smb)
    return out


# speedup vs baseline: 1.0066x; 1.0066x over previous
import math

import jax
import jax.numpy as jnp
from jax.experimental import pallas as pl
from jax.experimental.pallas import tpu as pltpu

T, B = 64, 8
DE = 100
HID = 200
NHID = 100
NLAYERS = 4
N_CLASSES = 6
LAMDA, ALPHA = 0.5, 0.1
N = T * B
GE = 128
DPP = 8
PI = math.pi

_F32 = jnp.float32


def _dot(a, b):
    return jax.lax.dot(a, b, preferred_element_type=_F32)


def _acos(x):
    a = jnp.abs(x)
    p = jnp.float32(-0.0187293)
    p = p * a + jnp.float32(0.0742610)
    p = p * a + jnp.float32(-0.2121144)
    p = p * a + jnp.float32(1.5707288)
    r = jnp.sqrt(jnp.maximum(1.0 - a, 0.0)) * p
    return jnp.where(x < 0, PI - r, r)


def _sim(c):
    return 1.0 - _acos(jnp.clip(c * 0.99999, -1.0, 1.0)) * (1.0 / PI)


def _encoder_body(u_ref, ua_ref, uv_ref, qm_ref,
                  lawT_ref, lab_ref, lvwT_ref, lvb_ref, llwT_ref, llb_ref,
                  wihT_ref, whhT_ref, bsum_ref, spk_ref,
                  fa_ref, fv_ref, fl_ref,
                  ul_ref, xf_ref, xb_ref, hsf_ref, hsb_ref, out0_ref):
    fa_ref[...] = _dot(ua_ref[...], lawT_ref[...]) + lab_ref[...]
    fv_ref[...] = _dot(uv_ref[...], lvwT_ref[...]) + lvb_ref[...]
    ul_ref[...] = _dot(u_ref[...], llwT_ref[...]) + llb_ref[...]

    for l in range(2):
        xin = ul_ref[...] if l == 0 else out0_ref[...]
        xf_ref[...] = _dot(xin, wihT_ref[l, 0]) + bsum_ref[l, 0]
        xb_ref[...] = _dot(xin, wihT_ref[l, 1]) + bsum_ref[l, 1]
        whf = whhT_ref[l, 0]
        whb = whhT_ref[l, 1]
        bf16 = jnp.bfloat16

        def step(t, carry):
            hf, cf, hb, cb = carry
            gf = xf_ref[pl.ds(t * B, B), :] + _dot(hf.astype(bf16), whf)
            i = jax.nn.sigmoid(gf[:, 0:GE])
            f = jax.nn.sigmoid(gf[:, GE:2 * GE])
            g = jnp.tanh(gf[:, 2 * GE:3 * GE])
            o = jax.nn.sigmoid(gf[:, 3 * GE:4 * GE])
            cf = f * cf + i * g
            hf = o * jnp.tanh(cf)
            hsf_ref[pl.ds(t * B, B), :] = hf

            tb = (T - 1) - t
            gb = xb_ref[pl.ds(tb * B, B), :] + _dot(hb.astype(bf16), whb)
            i = jax.nn.sigmoid(gb[:, 0:GE])
            f = jax.nn.sigmoid(gb[:, GE:2 * GE])
            g = jnp.tanh(gb[:, 2 * GE:3 * GE])
            o = jax.nn.sigmoid(gb[:, 3 * GE:4 * GE])
            cb = f * cb + i * g
            hb = o * jnp.tanh(cb)
            hsb_ref[pl.ds(tb * B, B), :] = hb
            return hf, cf, hb, cb

        z = jnp.zeros((B, GE), _F32)
        jax.lax.fori_loop(0, T, step, (z, z, z, z), unroll=16)
        out0_ref[:, 0:DE] = hsf_ref[:, 0:DE]
        out0_ref[:, DE:HID] = hsb_ref[:, 0:DE]

    q = qm_ref[...]
    sel = q[:, 1:2] > q[:, 0:1]
    emb = jnp.where(sel, spk_ref[1:2, :], spk_ref[0:1, :])
    fl_ref[...] = out0_ref[...] + emb


def _gcn_body(fa_ref, fv_ref, fl_ref, fcwT_ref, fcb_ref, conv_ref,
              wf_ref, wh_ref, smb_ref, out_ref, a_ref, x_ref):
    for d in range(DPP):
        r0 = d * T
        s0 = d * 3 * T
        fs = (fa_ref[pl.ds(r0, T), :], fv_ref[pl.ds(r0, T), :],
              fl_ref[pl.ds(r0, T), :])
        nx = []
        for m in range(3):
            x = fs[m]
            x_ref[pl.ds(s0 + T * m, T), :] = x
            inv = jax.lax.rsqrt(jnp.sum(x * x, axis=1, keepdims=True))
            nx.append(x * inv)

        for m in range(3):
            s = jax.lax.dot_general(nx[m], nx[m], (((1,), (1,)), ((), ())),
                                    preferred_element_type=_F32)
            a_ref[pl.ds(s0 + T * m, T), pl.ds(T * m, T)] = _sim(s)

        row = jax.lax.broadcasted_iota(jnp.int32, (T, T), 0)
        col = jax.lax.broadcasted_iota(jnp.int32, (T, T), 1)
        eye = row == col
        for m in range(3):
            for n in range(m + 1, 3):
                cs = jnp.sum(nx[m] * nx[n], axis=1, keepdims=True)
                tile = jnp.where(eye, _sim(cs), 0.0)
                a_ref[pl.ds(s0 + T * m, T), pl.ds(T * n, T)] = tile
                a_ref[pl.ds(s0 + T * n, T), pl.ds(T * m, T)] = tile

        araw = a_ref[pl.ds(s0, 3 * T), :]
        dcol = jax.lax.rsqrt(jnp.sum(araw, axis=1, keepdims=True))
        drow = jax.lax.rsqrt(jnp.sum(araw, axis=0, keepdims=True))
        a_ref[pl.ds(s0, 3 * T), :] = araw * dcol * drow

    for d in range(DPP):
        r0 = d * T
        s0 = d * 3 * T
        feats = x_ref[pl.ds(s0, 3 * T), :]
        h0 = jax.nn.relu(_dot(feats, fcwT_ref[...]) + fcb_ref[...])
        h = h0
        adj = a_ref[pl.ds(s0, 3 * T), :]
        for i in range(NLAYERS):
            theta = math.log(LAMDA / (i + 1) + 1.0)
            hi = _dot(adj, h)
            mm = (_dot(hi, conv_ref[i, 0:NHID, :])
                  + _dot(h0, conv_ref[i, NHID:2 * NHID, :]))
            r = (1.0 - ALPHA) * hi + ALPHA * h0
            h = jax.nn.relu(theta * mm + (1.0 - theta) * r)

        acc = smb_ref[...]
        for m in range(3):
            fm = jax.nn.relu(feats[T * m:T * (m + 1), :])
            hm = jax.nn.relu(h[T * m:T * (m + 1), :])
            acc = acc + _dot(fm, wf_ref[m]) + _dot(hm, wh_ref[m])
        mx = jnp.max(acc, axis=1, keepdims=True)
        sh = acc - mx
        lse = jnp.log(jnp.sum(jnp.exp(sh), axis=1, keepdims=True))
        out_ref[pl.ds(r0, T), :] = sh - lse


def kernel(U, qmask, U_a, U_v, seq_lengths, lin_a_w, lin_a_b, lin_v_w,
           lin_v_b, lin_l_w, lin_l_b, lstm_wih, lstm_whh, lstm_bih, lstm_bhh,
           spk_emb, gcn_fc_w, gcn_fc_b, conv_w, smax_w, smax_b):
    del seq_lengths

    u_flat = U.reshape(N, -1)
    ua_bt = U_a.transpose(1, 0, 2).reshape(N, -1)
    uv_bt = U_v.transpose(1, 0, 2).reshape(N, -1)
    qm_tb = qmask.reshape(N, 2)
    lawT = lin_a_w.T
    lvwT = lin_v_w.T
    llwT = lin_l_w.T
    lab = lin_a_b.reshape(1, HID)
    lvb = lin_v_b.reshape(1, HID)
    llb = lin_l_b.reshape(1, HID)
    def _pad_gates(w):
        lead = w.shape[:-1]
        w4 = w.reshape(lead + (4, DE))
        pad = [(0, 0)] * len(lead) + [(0, 0), (0, GE - DE)]
        return jnp.pad(w4, pad).reshape(lead + (4 * GE,))

    wihT = _pad_gates(lstm_wih.transpose(0, 1, 3, 2))
    whhT = _pad_gates(lstm_whh.transpose(0, 1, 3, 2))
    whhT = jnp.pad(whhT, ((0, 0), (0, 0), (0, GE - DE), (0, 0)))
    whhT = whhT.astype(jnp.bfloat16)
    bsum = _pad_gates((lstm_bih + lstm_bhh)).reshape(2, 2, 1, 4 * GE)
    fcwT = gcn_fc_w.T
    fcb = gcn_fc_b.reshape(1, NHID)
    smwT = smax_w.T
    wf = jnp.stack([smwT[300 * m:300 * m + HID] for m in range(3)])
    wh = jnp.stack([smwT[300 * m + HID:300 * (m + 1)] for m in range(3)])
    smb = smax_b.reshape(1, N_CLASSES)

    fa, fv, fl_tb = pl.pallas_call(
        _encoder_body,
        out_shape=[
            jax.ShapeDtypeStruct((N, HID), _F32),
            jax.ShapeDtypeStruct((N, HID), _F32),
            jax.ShapeDtypeStruct((N, HID), _F32),
        ],
        scratch_shapes=[
            pltpu.VMEM((N, HID), _F32),
            pltpu.VMEM((N, 4 * GE), _F32),
            pltpu.VMEM((N, 4 * GE), _F32),
            pltpu.VMEM((N, GE), _F32),
            pltpu.VMEM((N, GE), _F32),
            pltpu.VMEM((N, HID), _F32),
        ],
    )(u_flat, ua_bt, uv_bt, qm_tb, lawT, lab, lvwT, lvb, llwT, llb,
      wihT, whhT, bsum, spk_emb)

    fl = fl_tb.reshape(T, B, HID).transpose(1, 0, 2).reshape(N, HID)

    blk = lambda b: (b, 0)
    zero2 = lambda b: (0, 0)
    zero3 = lambda b: (0, 0, 0)
    out = pl.pallas_call(
        _gcn_body,
        grid=(B // DPP,),
        in_specs=[
            pl.BlockSpec((DPP * T, HID), blk),
            pl.BlockSpec((DPP * T, HID), blk),
            pl.BlockSpec((DPP * T, HID), blk),
            pl.BlockSpec((HID, NHID), zero2),
            pl.BlockSpec((1, NHID), zero2),
            pl.BlockSpec((NLAYERS, 2 * NHID, NHID), zero3),
            pl.BlockSpec((3, HID, N_CLASSES), zero3),
            pl.BlockSpec((3, NHID, N_CLASSES), zero3),
            pl.BlockSpec((1, N_CLASSES), zero2),
        ],
        out_specs=pl.BlockSpec((DPP * T, N_CLASSES), blk),
        out_shape=jax.ShapeDtypeStruct((N, N_CLASSES), _F32),
        compiler_params=pltpu.CompilerParams(
            dimension_semantics=("parallel",)),
        scratch_shapes=[
            pltpu.VMEM((DPP * 3 * T, 3 * T), _F32),
            pltpu.VMEM((DPP * 3 * T, HID), _F32),
        ],
    )(fa, fv, fl, fcwT, fcb, conv_w, wf, wh, smb)
    return out


# speedup vs baseline: 1.1120x; 1.1048x over previous
import math

import jax
import jax.numpy as jnp
import numpy as np
from jax.experimental import pallas as pl
from jax.experimental.pallas import tpu as pltpu

T, B = 64, 8
DE = 100
HID = 200
NHID = 100
NLAYERS = 4
N_CLASSES = 6
LAMDA, ALPHA = 0.5, 0.1
N = T * B
GE = 128
PI = math.pi

_F32 = jnp.float32


def _dot(a, b):
    return jax.lax.dot(a, b, preferred_element_type=_F32)


def _dot_nt(a, b):
    return jax.lax.dot_general(a, b, (((1,), (1,)), ((), ())),
                               preferred_element_type=_F32)


def _acos(x):
    a = jnp.abs(x)
    p = jnp.float32(-0.0187293)
    p = p * a + jnp.float32(0.0742610)
    p = p * a + jnp.float32(-0.2121144)
    p = p * a + jnp.float32(1.5707288)
    r = jnp.sqrt(jnp.maximum(1.0 - a, 0.0)) * p
    return jnp.where(x < 0, PI - r, r)


def _sim(c):
    return 1.0 - _acos(jnp.clip(c * 0.99999, -1.0, 1.0)) * (1.0 / PI)


def _body(u_ref, ua_ref, uv_ref, qm_ref,
          law_ref, lab_ref, lvw_ref, lvb_ref, llw_ref, llb_ref,
          wihT_ref, whhT_ref, bsum_ref, spk_ref, perm_ref,
          fcw_ref, fcb_ref, conv_ref, wf_ref, wh_ref, smb_ref,
          out_ref,
          ul_ref, xf_ref, xb_ref, hsf_ref, hsb_ref, out0_ref,
          fa_ref, fv_ref, fl_ref, a_ref, x_ref):
    fa_ref[...] = _dot_nt(ua_ref[...], law_ref[...]) + lab_ref[...]
    fv_ref[...] = _dot_nt(uv_ref[...], lvw_ref[...]) + lvb_ref[...]
    ul_ref[...] = _dot_nt(u_ref[...], llw_ref[...]) + llb_ref[...]

    for l in range(2):
        xin = ul_ref[...] if l == 0 else out0_ref[...]
        xf_ref[...] = _dot(xin, wihT_ref[l, 0]) + bsum_ref[l, 0]
        xb_ref[...] = _dot(xin, wihT_ref[l, 1]) + bsum_ref[l, 1]
        whf = whhT_ref[l, 0]
        whb = whhT_ref[l, 1]
        bf16 = jnp.bfloat16

        def step(t, carry):
            hf, cf, hb, cb = carry
            gf = xf_ref[pl.ds(t * B, B), :] + _dot(hf.astype(bf16), whf)
            i = jax.nn.sigmoid(gf[:, 0:GE])
            f = jax.nn.sigmoid(gf[:, GE:2 * GE])
            g = jnp.tanh(gf[:, 2 * GE:3 * GE])
            o = jax.nn.sigmoid(gf[:, 3 * GE:4 * GE])
            cf = f * cf + i * g
            hf = o * jnp.tanh(cf)
            hsf_ref[pl.ds(t * B, B), :] = hf

            tb = (T - 1) - t
            gb = xb_ref[pl.ds(tb * B, B), :] + _dot(hb.astype(bf16), whb)
            i = jax.nn.sigmoid(gb[:, 0:GE])
            f = jax.nn.sigmoid(gb[:, GE:2 * GE])
            g = jnp.tanh(gb[:, 2 * GE:3 * GE])
            o = jax.nn.sigmoid(gb[:, 3 * GE:4 * GE])
            cb = f * cb + i * g
            hb = o * jnp.tanh(cb)
            hsb_ref[pl.ds(tb * B, B), :] = hb
            return hf, cf, hb, cb

        z = jnp.zeros((B, GE), _F32)
        jax.lax.fori_loop(0, T, step, (z, z, z, z), unroll=16)
        out0_ref[:, 0:DE] = hsf_ref[:, 0:DE]
        out0_ref[:, DE:HID] = hsb_ref[:, 0:DE]

    q = qm_ref[...]
    sel = q[:, 1:2] > q[:, 0:1]
    emb = jnp.where(sel, spk_ref[1:2, :], spk_ref[0:1, :])
    fl_ref[...] = _dot(perm_ref[...], out0_ref[...] + emb)

    for d in range(B):
        r0 = d * T
        s0 = d * 3 * T
        fs = (fa_ref[pl.ds(r0, T), :], fv_ref[pl.ds(r0, T), :],
              fl_ref[pl.ds(r0, T), :])
        nx = []
        for m in range(3):
            x = fs[m]
            x_ref[pl.ds(s0 + T * m, T), :] = x
            inv = jax.lax.rsqrt(jnp.sum(x * x, axis=1, keepdims=True))
            nx.append(x * inv)

        for m in range(3):
            s = _dot_nt(nx[m], nx[m])
            a_ref[pl.ds(s0 + T * m, T), pl.ds(T * m, T)] = _sim(s)

        row = jax.lax.broadcasted_iota(jnp.int32, (T, T), 0)
        col = jax.lax.broadcasted_iota(jnp.int32, (T, T), 1)
        eye = row == col
        for m in range(3):
            for n in range(m + 1, 3):
                cs = jnp.sum(nx[m] * nx[n], axis=1, keepdims=True)
                tile = jnp.where(eye, _sim(cs), 0.0)
                a_ref[pl.ds(s0 + T * m, T), pl.ds(T * n, T)] = tile
                a_ref[pl.ds(s0 + T * n, T), pl.ds(T * m, T)] = tile

        araw = a_ref[pl.ds(s0, 3 * T), :]
        dcol = jax.lax.rsqrt(jnp.sum(araw, axis=1, keepdims=True))
        drow = jax.lax.rsqrt(jnp.sum(araw, axis=0, keepdims=True))
        a_ref[pl.ds(s0, 3 * T), :] = araw * dcol * drow

    for d in range(B):
        r0 = d * T
        s0 = d * 3 * T
        feats = x_ref[pl.ds(s0, 3 * T), :]
        h0 = jax.nn.relu(_dot_nt(feats, fcw_ref[...]) + fcb_ref[...])
        h = h0
        adj = a_ref[pl.ds(s0, 3 * T), :]
        for i in range(NLAYERS):
            theta = math.log(LAMDA / (i + 1) + 1.0)
            hi = _dot(adj, h)
            mm = (_dot(hi, conv_ref[i, 0:NHID, :])
                  + _dot(h0, conv_ref[i, NHID:2 * NHID, :]))
            r = (1.0 - ALPHA) * hi + ALPHA * h0
            h = jax.nn.relu(theta * mm + (1.0 - theta) * r)

        acc = smb_ref[...]
        for m in range(3):
            fm = jax.nn.relu(feats[T * m:T * (m + 1), :])
            hm = jax.nn.relu(h[T * m:T * (m + 1), :])
            acc = acc + _dot(fm, wf_ref[m]) + _dot(hm, wh_ref[m])
        mx = jnp.max(acc, axis=1, keepdims=True)
        sh = acc - mx
        lse = jnp.log(jnp.sum(jnp.exp(sh), axis=1, keepdims=True))
        out_ref[pl.ds(r0, T), :] = sh - lse


_PERM = np.zeros((N, N), dtype=np.float32)
for _b in range(B):
    for _t in range(T):
        _PERM[_b * T + _t, _t * B + _b] = 1.0


def kernel(U, qmask, U_a, U_v, seq_lengths, lin_a_w, lin_a_b, lin_v_w,
           lin_v_b, lin_l_w, lin_l_b, lstm_wih, lstm_whh, lstm_bih, lstm_bhh,
           spk_emb, gcn_fc_w, gcn_fc_b, conv_w, smax_w, smax_b):
    del seq_lengths

    u_flat = U.reshape(N, -1)
    ua_bt = U_a.transpose(1, 0, 2).reshape(N, -1)
    uv_bt = U_v.transpose(1, 0, 2).reshape(N, -1)
    qm_tb = qmask.reshape(N, 2)
    lab = lin_a_b.reshape(1, HID)
    lvb = lin_v_b.reshape(1, HID)
    llb = lin_l_b.reshape(1, HID)

    def _pad_gates(w):
        lead = w.shape[:-1]
        w4 = w.reshape(lead + (4, DE))
        pad = [(0, 0)] * len(lead) + [(0, 0), (0, GE - DE)]
        return jnp.pad(w4, pad).reshape(lead + (4 * GE,))

    wihT = _pad_gates(lstm_wih.transpose(0, 1, 3, 2))
    whhT = _pad_gates(lstm_whh.transpose(0, 1, 3, 2))
    whhT = jnp.pad(whhT, ((0, 0), (0, 0), (0, GE - DE), (0, 0)))
    whhT = whhT.astype(jnp.bfloat16)
    bsum = _pad_gates((lstm_bih + lstm_bhh)).reshape(2, 2, 1, 4 * GE)
    fcb = gcn_fc_b.reshape(1, NHID)
    smwT = smax_w.T
    wf = jnp.stack([smwT[300 * m:300 * m + HID] for m in range(3)])
    wh = jnp.stack([smwT[300 * m + HID:300 * (m + 1)] for m in range(3)])
    smb = smax_b.reshape(1, N_CLASSES)
    perm = jnp.asarray(_PERM)

    out = pl.pallas_call(
        _body,
        out_shape=jax.ShapeDtypeStruct((N, N_CLASSES), _F32),
        scratch_shapes=[
            pltpu.VMEM((N, HID), _F32),
            pltpu.VMEM((N, 4 * GE), _F32),
            pltpu.VMEM((N, 4 * GE), _F32),
            pltpu.VMEM((N, GE), _F32),
            pltpu.VMEM((N, GE), _F32),
            pltpu.VMEM((N, HID), _F32),
            pltpu.VMEM((N, HID), _F32),
            pltpu.VMEM((N, HID), _F32),
            pltpu.VMEM((N, HID), _F32),
            pltpu.VMEM((B * 3 * T, 3 * T), _F32),
            pltpu.VMEM((B * 3 * T, HID), _F32),
        ],
    )(u_flat, ua_bt, uv_bt, qm_tb, lin_a_w, lab, lin_v_w, lvb, lin_l_w, llb,
      wihT, whhT, bsum, spk_emb, perm, gcn_fc_w, fcb, conv_w, wf, wh, smb)
    return out


# speedup vs baseline: 1.2796x; 1.1507x over previous
import math

import jax
import jax.numpy as jnp
import numpy as np
from jax.experimental import pallas as pl
from jax.experimental.pallas import tpu as pltpu

T, B = 64, 8
DE = 100
HID = 200
NHID = 100
NLAYERS = 4
N_CLASSES = 6
LAMDA, ALPHA = 0.5, 0.1
N = T * B
GE = 128
PI = math.pi

_F32 = jnp.float32


def _dot(a, b):
    return jax.lax.dot(a, b, preferred_element_type=_F32)


def _dot_nt(a, b):
    return jax.lax.dot_general(a, b, (((1,), (1,)), ((), ())),
                               preferred_element_type=_F32)


def _acos(x):
    a = jnp.abs(x)
    p = jnp.float32(-0.0187293)
    p = p * a + jnp.float32(0.0742610)
    p = p * a + jnp.float32(-0.2121144)
    p = p * a + jnp.float32(1.5707288)
    r = jnp.sqrt(jnp.maximum(1.0 - a, 0.0)) * p
    return jnp.where(x < 0, PI - r, r)


def _sim(c):
    return 1.0 - _acos(jnp.clip(c * 0.99999, -1.0, 1.0)) * (1.0 / PI)


def _body(u_ref, ua_ref, uv_ref, qm_ref,
          law_ref, lab_ref, lvw_ref, lvb_ref, llw_ref, llb_ref,
          wihT_ref, whhT_ref, bsum_ref, spk_ref, perm_ref,
          fcw_ref, fcb_ref, conv_ref, wfb_ref, whb_ref, smb_ref,
          out_ref,
          ul_ref, xf_ref, xb_ref, hsf_ref, hsb_ref, out0_ref,
          fa_ref, fv_ref, fl_ref, a_ref, x_ref, nx_ref,
          h0_ref, h_ref, l1_ref, l2_ref):
    fa_ref[...] = _dot_nt(ua_ref[...], law_ref[...]) + lab_ref[...]
    fv_ref[...] = _dot_nt(uv_ref[...], lvw_ref[...]) + lvb_ref[...]
    ul_ref[...] = _dot_nt(u_ref[...], llw_ref[...]) + llb_ref[...]

    for l in range(2):
        xin = ul_ref[...] if l == 0 else out0_ref[...]
        xf_ref[...] = _dot(xin, wihT_ref[l, 0]) + bsum_ref[l, 0]
        xb_ref[...] = _dot(xin, wihT_ref[l, 1]) + bsum_ref[l, 1]
        whf = whhT_ref[l, 0]
        whb = whhT_ref[l, 1]
        bf16 = jnp.bfloat16

        def step(t, carry):
            hf, cf, hb, cb = carry
            gf = xf_ref[pl.ds(t * B, B), :] + _dot(hf.astype(bf16), whf)
            i = jax.nn.sigmoid(gf[:, 0:GE])
            f = jax.nn.sigmoid(gf[:, GE:2 * GE])
            g = jnp.tanh(gf[:, 2 * GE:3 * GE])
            o = jax.nn.sigmoid(gf[:, 3 * GE:4 * GE])
            cf = f * cf + i * g
            hf = o * jnp.tanh(cf)
            hsf_ref[pl.ds(t * B, B), :] = hf

            tb = (T - 1) - t
            gb = xb_ref[pl.ds(tb * B, B), :] + _dot(hb.astype(bf16), whb)
            i = jax.nn.sigmoid(gb[:, 0:GE])
            f = jax.nn.sigmoid(gb[:, GE:2 * GE])
            g = jnp.tanh(gb[:, 2 * GE:3 * GE])
            o = jax.nn.sigmoid(gb[:, 3 * GE:4 * GE])
            cb = f * cb + i * g
            hb = o * jnp.tanh(cb)
            hsb_ref[pl.ds(tb * B, B), :] = hb
            return hf, cf, hb, cb

        z = jnp.zeros((B, GE), _F32)
        jax.lax.fori_loop(0, T, step, (z, z, z, z), unroll=16)
        out0_ref[:, 0:DE] = hsf_ref[:, 0:DE]
        out0_ref[:, DE:HID] = hsb_ref[:, 0:DE]

    q = qm_ref[...]
    sel = q[:, 1:2] > q[:, 0:1]
    emb = jnp.where(sel, spk_ref[1:2, :], spk_ref[0:1, :])
    fl_ref[...] = _dot(perm_ref[...], out0_ref[...] + emb)

    for d in range(B):
        r0 = d * T
        s0 = d * 3 * T
        fs = (fa_ref[pl.ds(r0, T), :], fv_ref[pl.ds(r0, T), :],
              fl_ref[pl.ds(r0, T), :])
        for m in range(3):
            x = fs[m]
            x_ref[pl.ds(s0 + T * m, T), :] = x
            inv = jax.lax.rsqrt(jnp.sum(x * x, axis=1, keepdims=True))
            nx_ref[pl.ds(s0 + T * m, T), :] = x * inv

    for d in range(B):
        s0 = d * 3 * T
        for m in range(3):
            nxm = nx_ref[pl.ds(s0 + T * m, T), :]
            s = _dot_nt(nxm, nxm)
            a_ref[pl.ds(s0 + T * m, T), pl.ds(T * m, T)] = _sim(s)

    row = jax.lax.broadcasted_iota(jnp.int32, (T, T), 0)
    col = jax.lax.broadcasted_iota(jnp.int32, (T, T), 1)
    eye = row == col
    for d in range(B):
        s0 = d * 3 * T
        for m in range(3):
            for n in range(m + 1, 3):
                nxm = nx_ref[pl.ds(s0 + T * m, T), :]
                nxn = nx_ref[pl.ds(s0 + T * n, T), :]
                cs = jnp.sum(nxm * nxn, axis=1, keepdims=True)
                tile = jnp.where(eye, _sim(cs), 0.0)
                a_ref[pl.ds(s0 + T * m, T), pl.ds(T * n, T)] = tile
                a_ref[pl.ds(s0 + T * n, T), pl.ds(T * m, T)] = tile

    for d in range(B):
        s0 = d * 3 * T
        araw = a_ref[pl.ds(s0, 3 * T), :]
        dcol = jax.lax.rsqrt(jnp.sum(araw, axis=1, keepdims=True))
        drow = jax.lax.rsqrt(jnp.sum(araw, axis=0, keepdims=True))
        a_ref[pl.ds(s0, 3 * T), :] = araw * dcol * drow

    h0_all = jax.nn.relu(_dot_nt(x_ref[...], fcw_ref[...]) + fcb_ref[...])
    h0_ref[...] = h0_all
    h_ref[...] = h0_all
    for i in range(NLAYERS):
        theta = math.log(LAMDA / (i + 1) + 1.0)
        for d in range(B):
            s0 = d * 3 * T
            adj = a_ref[pl.ds(s0, 3 * T), :]
            h = h_ref[pl.ds(s0, 3 * T), :]
            h0 = h0_ref[pl.ds(s0, 3 * T), :]
            hi = _dot(adj, h)
            mm = (_dot(hi, conv_ref[i, 0:NHID, :])
                  + _dot(h0, conv_ref[i, NHID:2 * NHID, :]))
            r = (1.0 - ALPHA) * hi + ALPHA * h0
            h_ref[pl.ds(s0, 3 * T), :] = jax.nn.relu(theta * mm
                                                     + (1.0 - theta) * r)

    l1_ref[...] = _dot(jax.nn.relu(x_ref[...]), wfb_ref[...])
    l2_ref[...] = _dot(jax.nn.relu(h_ref[...]), whb_ref[...])
    for d in range(B):
        s0 = d * 3 * T
        acc = smb_ref[...] + jnp.zeros((T, N_CLASSES), _F32)
        for m in range(3):
            c0 = N_CLASSES * m
            acc = acc + l1_ref[pl.ds(s0 + T * m, T), pl.ds(c0, N_CLASSES)]
            acc = acc + l2_ref[pl.ds(s0 + T * m, T), pl.ds(c0, N_CLASSES)]
        mx = jnp.max(acc, axis=1, keepdims=True)
        sh = acc - mx
        lse = jnp.log(jnp.sum(jnp.exp(sh), axis=1, keepdims=True))
        out_ref[pl.ds(d * T, T), :] = sh - lse


_PERM = np.zeros((N, N), dtype=np.float32)
for _b in range(B):
    for _t in range(T):
        _PERM[_b * T + _t, _t * B + _b] = 1.0


def kernel(U, qmask, U_a, U_v, seq_lengths, lin_a_w, lin_a_b, lin_v_w,
           lin_v_b, lin_l_w, lin_l_b, lstm_wih, lstm_whh, lstm_bih, lstm_bhh,
           spk_emb, gcn_fc_w, gcn_fc_b, conv_w, smax_w, smax_b):
    del seq_lengths

    u_flat = U.reshape(N, -1)
    ua_bt = U_a.transpose(1, 0, 2).reshape(N, -1)
    uv_bt = U_v.transpose(1, 0, 2).reshape(N, -1)
    qm_tb = qmask.reshape(N, 2)
    lab = lin_a_b.reshape(1, HID)
    lvb = lin_v_b.reshape(1, HID)
    llb = lin_l_b.reshape(1, HID)

    def _pad_gates(w):
        lead = w.shape[:-1]
        w4 = w.reshape(lead + (4, DE))
        pad = [(0, 0)] * len(lead) + [(0, 0), (0, GE - DE)]
        return jnp.pad(w4, pad).reshape(lead + (4 * GE,))

    wihT = _pad_gates(lstm_wih.transpose(0, 1, 3, 2))
    whhT = _pad_gates(lstm_whh.transpose(0, 1, 3, 2))
    whhT = jnp.pad(whhT, ((0, 0), (0, 0), (0, GE - DE), (0, 0)))
    whhT = whhT.astype(jnp.bfloat16)
    bsum = _pad_gates((lstm_bih + lstm_bhh)).reshape(2, 2, 1, 4 * GE)
    fcb = gcn_fc_b.reshape(1, NHID)
    smwT = smax_w.T
    wfb = jnp.concatenate(
        [smwT[300 * m:300 * m + HID] for m in range(3)], axis=1)
    whb = jnp.concatenate(
        [smwT[300 * m + HID:300 * (m + 1)] for m in range(3)], axis=1)
    smb = smax_b.reshape(1, N_CLASSES)
    perm = jnp.asarray(_PERM)

    out = pl.pallas_call(
        _body,
        out_shape=jax.ShapeDtypeStruct((N, N_CLASSES), _F32),
        scratch_shapes=[
            pltpu.VMEM((N, HID), _F32),
            pltpu.VMEM((N, 4 * GE), _F32),
            pltpu.VMEM((N, 4 * GE), _F32),
            pltpu.VMEM((N, GE), _F32),
            pltpu.VMEM((N, GE), _F32),
            pltpu.VMEM((N, HID), _F32),
            pltpu.VMEM((N, HID), _F32),
            pltpu.VMEM((N, HID), _F32),
            pltpu.VMEM((N, HID), _F32),
            pltpu.VMEM((B * 3 * T, 3 * T), _F32),
            pltpu.VMEM((B * 3 * T, HID), _F32),
            pltpu.VMEM((B * 3 * T, HID), _F32),
            pltpu.VMEM((B * 3 * T, NHID), _F32),
            pltpu.VMEM((B * 3 * T, NHID), _F32),
            pltpu.VMEM((B * 3 * T, 3 * N_CLASSES), _F32),
            pltpu.VMEM((B * 3 * T, 3 * N_CLASSES), _F32),
        ],
    )(u_flat, ua_bt, uv_bt, qm_tb, lin_a_w, lab, lin_v_w, lvb, lin_l_w, llb,
      wihT, whhT, bsum, spk_emb, perm, gcn_fc_w, fcb, conv_w, wfb, whb, smb)
    return out


# speedup vs baseline: 1.3082x; 1.0224x over previous
import math

import jax
import jax.numpy as jnp
import numpy as np
from jax.experimental import pallas as pl
from jax.experimental.pallas import tpu as pltpu

T, B = 64, 8
DE = 100
HID = 200
NHID = 100
NLAYERS = 4
N_CLASSES = 6
LAMDA, ALPHA = 0.5, 0.1
N = T * B
GE = 128
PI = math.pi

_F32 = jnp.float32


def _dot(a, b):
    return jax.lax.dot(a, b, preferred_element_type=_F32)


def _dot_nt(a, b):
    return jax.lax.dot_general(a, b, (((1,), (1,)), ((), ())),
                               preferred_element_type=_F32)


def _acos(x):
    a = jnp.abs(x)
    p = jnp.float32(-0.0187293)
    p = p * a + jnp.float32(0.0742610)
    p = p * a + jnp.float32(-0.2121144)
    p = p * a + jnp.float32(1.5707288)
    r = jnp.sqrt(jnp.maximum(1.0 - a, 0.0)) * p
    return jnp.where(x < 0, PI - r, r)


def _sim(c):
    return 1.0 - _acos(jnp.clip(c * 0.99999, -1.0, 1.0)) * (1.0 / PI)


def _body(u_ref, ua_ref, uv_ref, qm_ref,
          law_ref, lab_ref, lvw_ref, lvb_ref, llw_ref, llb_ref,
          wihT_ref, whhT_ref, bsum_ref, spk_ref, perm_ref,
          fcw_ref, fcb_ref, conv_ref, wfb_ref, whb_ref, smb_ref,
          out_ref,
          ul_ref, xf_ref, xb_ref, hsf_ref, hsb_ref, out0_ref,
          fa_ref, fv_ref, fl_ref, a_ref, x_ref, nx_ref,
          h0_ref, h_ref, l1_ref, l2_ref):
    fa_ref[...] = _dot_nt(ua_ref[...], law_ref[...]) + lab_ref[...]
    fv_ref[...] = _dot_nt(uv_ref[...], lvw_ref[...]) + lvb_ref[...]
    ul_ref[...] = _dot_nt(u_ref[...], llw_ref[...]) + llb_ref[...]

    row = jax.lax.broadcasted_iota(jnp.int32, (T, T), 0)
    col = jax.lax.broadcasted_iota(jnp.int32, (T, T), 1)
    eye = row == col

    def _norm_item(src_ref, d, m):
        s0 = d * 3 * T
        x = src_ref[pl.ds(d * T, T), :]
        x_ref[pl.ds(s0 + T * m, T), :] = x
        inv = jax.lax.rsqrt(jnp.sum(x * x, axis=1, keepdims=True))
        nx_ref[pl.ds(s0 + T * m, T), :] = x * inv

    def _gram_item(d, m):
        s0 = d * 3 * T
        nxm = nx_ref[pl.ds(s0 + T * m, T), :]
        a_ref[pl.ds(s0 + T * m, T), pl.ds(T * m, T)] = _sim(_dot_nt(nxm, nxm))

    def _cross_item(d, m, n):
        s0 = d * 3 * T
        nxm = nx_ref[pl.ds(s0 + T * m, T), :]
        nxn = nx_ref[pl.ds(s0 + T * n, T), :]
        cs = jnp.sum(nxm * nxn, axis=1, keepdims=True)
        tile = jnp.where(eye, _sim(cs), 0.0)
        a_ref[pl.ds(s0 + T * m, T), pl.ds(T * n, T)] = tile
        a_ref[pl.ds(s0 + T * n, T), pl.ds(T * m, T)] = tile

    av_work = []
    for d in range(B):
        av_work.append(lambda d=d: _norm_item(fa_ref, d, 0))
        av_work.append(lambda d=d: _norm_item(fv_ref, d, 1))
    for d in range(B):
        av_work.append(lambda d=d: _gram_item(d, 0))
        av_work.append(lambda d=d: _gram_item(d, 1))
    for d in range(B):
        av_work.append(lambda d=d: _cross_item(d, 0, 1))
    wq = iter(av_work)

    for l in range(2):
        xin = ul_ref[...] if l == 0 else out0_ref[...]
        xf_ref[...] = _dot(xin, wihT_ref[l, 0]) + bsum_ref[l, 0]
        xb_ref[...] = _dot(xin, wihT_ref[l, 1]) + bsum_ref[l, 1]
        whf = whhT_ref[l, 0]
        whb = whhT_ref[l, 1]
        bf16 = jnp.bfloat16

        z = jnp.zeros((B, GE), _F32)
        hf, cf, hb, cb = z, z, z, z
        for t in range(T):
            gf = xf_ref[pl.ds(t * B, B), :] + _dot(hf.astype(bf16), whf)
            i = jax.nn.sigmoid(gf[:, 0:GE])
            f = jax.nn.sigmoid(gf[:, GE:2 * GE])
            g = jnp.tanh(gf[:, 2 * GE:3 * GE])
            o = jax.nn.sigmoid(gf[:, 3 * GE:4 * GE])
            cf = f * cf + i * g
            hf = o * jnp.tanh(cf)
            hsf_ref[pl.ds(t * B, B), :] = hf

            tb = (T - 1) - t
            gb = xb_ref[pl.ds(tb * B, B), :] + _dot(hb.astype(bf16), whb)
            i = jax.nn.sigmoid(gb[:, 0:GE])
            f = jax.nn.sigmoid(gb[:, GE:2 * GE])
            g = jnp.tanh(gb[:, 2 * GE:3 * GE])
            o = jax.nn.sigmoid(gb[:, 3 * GE:4 * GE])
            cb = f * cb + i * g
            hb = o * jnp.tanh(cb)
            hsb_ref[pl.ds(tb * B, B), :] = hb

            if t % 3 == 2:
                item = next(wq, None)
                if item is not None:
                    item()
        out0_ref[:, 0:DE] = hsf_ref[:, 0:DE]
        out0_ref[:, DE:HID] = hsb_ref[:, 0:DE]

    q = qm_ref[...]
    sel = q[:, 1:2] > q[:, 0:1]
    emb = jnp.where(sel, spk_ref[1:2, :], spk_ref[0:1, :])
    fl_ref[...] = _dot(perm_ref[...], out0_ref[...] + emb)

    for d in range(B):
        _norm_item(fl_ref, d, 2)
    for d in range(B):
        _gram_item(d, 2)
    for d in range(B):
        _cross_item(d, 0, 2)
        _cross_item(d, 1, 2)

    for d in range(B):
        s0 = d * 3 * T
        araw = a_ref[pl.ds(s0, 3 * T), :]
        dcol = jax.lax.rsqrt(jnp.sum(araw, axis=1, keepdims=True))
        drow = jax.lax.rsqrt(jnp.sum(araw, axis=0, keepdims=True))
        a_ref[pl.ds(s0, 3 * T), :] = araw * dcol * drow

    h0_all = jax.nn.relu(_dot_nt(x_ref[...], fcw_ref[...]) + fcb_ref[...])
    h0_ref[...] = h0_all
    h_ref[...] = h0_all
    for i in range(NLAYERS):
        theta = math.log(LAMDA / (i + 1) + 1.0)
        for d in range(B):
            s0 = d * 3 * T
            adj = a_ref[pl.ds(s0, 3 * T), :]
            h = h_ref[pl.ds(s0, 3 * T), :]
            h0 = h0_ref[pl.ds(s0, 3 * T), :]
            hi = _dot(adj, h)
            mm = (_dot(hi, conv_ref[i, 0:NHID, :])
                  + _dot(h0, conv_ref[i, NHID:2 * NHID, :]))
            r = (1.0 - ALPHA) * hi + ALPHA * h0
            h_ref[pl.ds(s0, 3 * T), :] = jax.nn.relu(theta * mm
                                                     + (1.0 - theta) * r)

    l1_ref[...] = _dot(jax.nn.relu(x_ref[...]), wfb_ref[...])
    l2_ref[...] = _dot(jax.nn.relu(h_ref[...]), whb_ref[...])
    for d in range(B):
        s0 = d * 3 * T
        acc = smb_ref[...] + jnp.zeros((T, N_CLASSES), _F32)
        for m in range(3):
            c0 = N_CLASSES * m
            acc = acc + l1_ref[pl.ds(s0 + T * m, T), pl.ds(c0, N_CLASSES)]
            acc = acc + l2_ref[pl.ds(s0 + T * m, T), pl.ds(c0, N_CLASSES)]
        mx = jnp.max(acc, axis=1, keepdims=True)
        sh = acc - mx
        lse = jnp.log(jnp.sum(jnp.exp(sh), axis=1, keepdims=True))
        out_ref[pl.ds(d * T, T), :] = sh - lse


_PERM = np.zeros((N, N), dtype=np.float32)
for _b in range(B):
    for _t in range(T):
        _PERM[_b * T + _t, _t * B + _b] = 1.0


def kernel(U, qmask, U_a, U_v, seq_lengths, lin_a_w, lin_a_b, lin_v_w,
           lin_v_b, lin_l_w, lin_l_b, lstm_wih, lstm_whh, lstm_bih, lstm_bhh,
           spk_emb, gcn_fc_w, gcn_fc_b, conv_w, smax_w, smax_b):
    del seq_lengths

    u_flat = U.reshape(N, -1)
    ua_bt = U_a.transpose(1, 0, 2).reshape(N, -1)
    uv_bt = U_v.transpose(1, 0, 2).reshape(N, -1)
    qm_tb = qmask.reshape(N, 2)
    lab = lin_a_b.reshape(1, HID)
    lvb = lin_v_b.reshape(1, HID)
    llb = lin_l_b.reshape(1, HID)

    def _pad_gates(w):
        lead = w.shape[:-1]
        w4 = w.reshape(lead + (4, DE))
        pad = [(0, 0)] * len(lead) + [(0, 0), (0, GE - DE)]
        return jnp.pad(w4, pad).reshape(lead + (4 * GE,))

    wihT = _pad_gates(lstm_wih.transpose(0, 1, 3, 2))
    whhT = _pad_gates(lstm_whh.transpose(0, 1, 3, 2))
    whhT = jnp.pad(whhT, ((0, 0), (0, 0), (0, GE - DE), (0, 0)))
    whhT = whhT.astype(jnp.bfloat16)
    bsum = _pad_gates((lstm_bih + lstm_bhh)).reshape(2, 2, 1, 4 * GE)
    fcb = gcn_fc_b.reshape(1, NHID)
    smwT = smax_w.T
    wfb = jnp.concatenate(
        [smwT[300 * m:300 * m + HID] for m in range(3)], axis=1)
    whb = jnp.concatenate(
        [smwT[300 * m + HID:300 * (m + 1)] for m in range(3)], axis=1)
    smb = smax_b.reshape(1, N_CLASSES)
    perm = jnp.asarray(_PERM)

    out = pl.pallas_call(
        _body,
        out_shape=jax.ShapeDtypeStruct((N, N_CLASSES), _F32),
        scratch_shapes=[
            pltpu.VMEM((N, HID), _F32),
            pltpu.VMEM((N, 4 * GE), _F32),
            pltpu.VMEM((N, 4 * GE), _F32),
            pltpu.VMEM((N, GE), _F32),
            pltpu.VMEM((N, GE), _F32),
            pltpu.VMEM((N, HID), _F32),
            pltpu.VMEM((N, HID), _F32),
            pltpu.VMEM((N, HID), _F32),
            pltpu.VMEM((N, HID), _F32),
            pltpu.VMEM((B * 3 * T, 3 * T), _F32),
            pltpu.VMEM((B * 3 * T, HID), _F32),
            pltpu.VMEM((B * 3 * T, HID), _F32),
            pltpu.VMEM((B * 3 * T, NHID), _F32),
            pltpu.VMEM((B * 3 * T, NHID), _F32),
            pltpu.VMEM((B * 3 * T, 3 * N_CLASSES), _F32),
            pltpu.VMEM((B * 3 * T, 3 * N_CLASSES), _F32),
        ],
    )(u_flat, ua_bt, uv_bt, qm_tb, lin_a_w, lab, lin_v_w, lvb, lin_l_w, llb,
      wihT, whhT, bsum, spk_emb, perm, gcn_fc_w, fcb, conv_w, wfb, whb, smb)
    return out
